# Initial kernel scaffold; baseline (speedup 1.0000x reference)
import jax
import jax.numpy as jnp
from jax.experimental import pallas as pl


def kernel(pred, target, batch_size):
    raise NotImplementedError("write your pallas kernel here")


# speedup vs baseline: 1.0651x; 1.0651x over previous
import functools

import jax
import jax.numpy as jnp
from jax import lax
from jax.experimental import pallas as pl
from jax.experimental.pallas import tpu as pltpu
from jax.experimental.pallas import tpu_sc as plsc

_BINS = 30
_ROWS = 16384
_COLS = 1024
_TOTAL = _ROWS * _COLS

_NC = 2
_NS = 16
_L = 16
_NW = _NC * _NS
_PER_W = _TOTAL // _NW
_CH = 16384
_NCHUNK = _PER_W // _CH

_BR = 512


def _prep_body(p_ref, t_ref, idx_ref, l_ref):
    p = p_ref[...]
    t = t_ref[...]
    g = jnp.abs(p - t)
    binf = jnp.minimum(jnp.floor(g * _BINS), _BINS - 1)
    idx_ref[...] = binf.astype(jnp.int32)
    log_p = jnp.maximum(jnp.log(p), -100.0)
    log_1mp = jnp.maximum(jnp.log(1.0 - p), -100.0)
    l_ref[...] = -(t * log_p + (1.0 - t) * log_1mp)


_prep = pl.pallas_call(
    _prep_body,
    grid=(_ROWS // _BR,),
    in_specs=[pl.BlockSpec((_BR, _COLS), lambda i: (i, 0))] * 2,
    out_specs=[pl.BlockSpec((_BR, _COLS), lambda i: (i, 0))] * 2,
    out_shape=[
        jax.ShapeDtypeStruct((_ROWS, _COLS), jnp.int32),
        jax.ShapeDtypeStruct((_ROWS, _COLS), jnp.float32),
    ],
)


_PB = 32
_ACC = _L * _PB


@functools.partial(
    pl.kernel,
    mesh=plsc.VectorSubcoreMesh(core_axis_name="c", subcore_axis_name="s"),
    compiler_params=pltpu.CompilerParams(needs_layout_passes=False),
    out_type=(
        jax.ShapeDtypeStruct((_NW * _ACC,), jnp.float32),
        jax.ShapeDtypeStruct((_NW * _ACC,), jnp.float32),
    ),
    scratch_types=[
        pltpu.VMEM((_CH,), jnp.int32),
        pltpu.VMEM((_CH,), jnp.float32),
        pltpu.VMEM((_ACC,), jnp.float32),
        pltpu.VMEM((_ACC,), jnp.float32),
    ],
)
def _sc_hist(idx_hbm, l_hbm, cnt_out, sum_out, idx_v, l_v, cnt_acc, sum_acc):
    wid = lax.axis_index("s") * _NC + lax.axis_index("c")
    base = wid * _PER_W
    lane_off = lax.iota(jnp.int32, _L) * _PB
    ones = jnp.ones((_L,), jnp.float32)
    zeros = jnp.zeros((_L,), jnp.float32)
    for v in range(_ACC // _L):
        cnt_acc[pl.ds(v * _L, _L)] = zeros
        sum_acc[pl.ds(v * _L, _L)] = zeros

    def chunk_body(c, carry):
        off = base + c * _CH
        pltpu.sync_copy(idx_hbm.at[pl.ds(off, _CH)], idx_v)
        pltpu.sync_copy(l_hbm.at[pl.ds(off, _CH)], l_v)

        def vec_body(i, carry2):
            bv = idx_v[pl.ds(i * _L, _L)] + lane_off
            lv = l_v[pl.ds(i * _L, _L)]
            plsc.addupdate_scatter(cnt_acc, [bv], ones)
            plsc.addupdate_scatter(sum_acc, [bv], lv)
            return carry2

        return lax.fori_loop(0, _CH // _L, vec_body, carry)

    lax.fori_loop(0, _NCHUNK, chunk_body, 0)
    pltpu.sync_copy(cnt_acc, cnt_out.at[pl.ds(wid * _ACC, _ACC)])
    pltpu.sync_copy(sum_acc, sum_out.at[pl.ds(wid * _ACC, _ACC)])


def _combine_body(cnt_ref, sum_ref, out_ref):
    cnt = jnp.sum(cnt_ref[...], axis=0, keepdims=True)
    s = jnp.sum(sum_ref[...], axis=0, keepdims=True)
    nonempty = cnt > 0.0
    n = jnp.sum(nonempty.astype(jnp.float32))
    terms = jnp.where(nonempty, s / jnp.maximum(cnt, 1.0), 0.0)
    out_ref[0, 0] = jnp.where(n > 0.0, jnp.sum(terms) / jnp.maximum(n, 1.0), 0.0)


_combine = pl.pallas_call(
    _combine_body,
    in_specs=[
        pl.BlockSpec((_NW * _L, _PB), lambda: (0, 0)),
        pl.BlockSpec((_NW * _L, _PB), lambda: (0, 0)),
    ],
    out_specs=pl.BlockSpec(memory_space=pltpu.SMEM),
    out_shape=jax.ShapeDtypeStruct((1, 1), jnp.float32),
)


def kernel(pred, target, batch_size):
    idx, l = _prep(pred, target)
    cnt, s = _sc_hist(idx.reshape(_TOTAL), l.reshape(_TOTAL))
    out = _combine(cnt.reshape(_NW * _L, _PB), s.reshape(_NW * _L, _PB))
    return out[0, 0]


# speedup vs baseline: 1.4596x; 1.3703x over previous
import functools

import jax
import jax.numpy as jnp
from jax import lax
from jax.experimental import pallas as pl
from jax.experimental.pallas import tpu as pltpu
from jax.experimental.pallas import tpu_sc as plsc

_BINS = 30
_ROWS = 16384
_COLS = 1024
_TOTAL = _ROWS * _COLS

_NC = 2
_NS = 16
_L = 16
_NW = _NC * _NS
_WROWS = _ROWS // _NW
_CR = 32
_NCHUNK = _WROWS // _CR
_VPC = _CR * _COLS // _L
_UNROLL = 8

_BR = 512


def _prep_body(p_ref, t_ref, w_ref):
    p = p_ref[...]
    t = t_ref[...]
    g = jnp.abs(p - t)
    binv = jnp.minimum(jnp.floor(g * _BINS), _BINS - 1).astype(jnp.uint32)
    log_p = jnp.maximum(jnp.log(p), -100.0)
    log_1mp = jnp.maximum(jnp.log(1.0 - p), -100.0)
    l = -(t * log_p + (1.0 - t) * log_1mp)
    bits = lax.bitcast_convert_type(l, jnp.uint32) + jnp.uint32(0x8000)
    word = (binv << jnp.uint32(16)) | lax.shift_right_logical(bits, jnp.uint32(16))
    w_ref[...] = lax.bitcast_convert_type(word, jnp.int32)


_prep = pl.pallas_call(
    _prep_body,
    grid=(_ROWS // _BR,),
    in_specs=[pl.BlockSpec((_BR, _COLS), lambda i: (i, 0))] * 2,
    out_specs=pl.BlockSpec((_BR, _COLS), lambda i: (i, 0)),
    out_shape=jax.ShapeDtypeStruct((_ROWS, _COLS), jnp.int32),
)


_PB = 32
_ACC = _L * _PB


@functools.partial(
    pl.kernel,
    mesh=plsc.VectorSubcoreMesh(core_axis_name="c", subcore_axis_name="s"),
    compiler_params=pltpu.CompilerParams(needs_layout_passes=False),
    out_type=(
        jax.ShapeDtypeStruct((_NW * _ACC,), jnp.float32),
        jax.ShapeDtypeStruct((_NW * _ACC,), jnp.float32),
    ),
    scratch_types=[
        pltpu.VMEM((2, _CR, _COLS), jnp.int32),
        pltpu.VMEM((_ACC,), jnp.float32),
        pltpu.VMEM((_ACC,), jnp.float32),
        pltpu.SemaphoreType.DMA,
        pltpu.SemaphoreType.DMA,
    ],
)
def _sc_hist(w_hbm, cnt_out, sum_out, buf, cnt_acc, sum_acc, sem0, sem1):
    wid = lax.axis_index("s") * _NC + lax.axis_index("c")
    row0 = wid * _WROWS
    lane_off = lax.iota(jnp.int32, _L) * _PB
    ones = jnp.ones((_L,), jnp.float32)
    zeros = jnp.zeros((_L,), jnp.float32)
    sems = (sem0, sem1)
    for v in range(_ACC // _L):
        cnt_acc[pl.ds(v * _L, _L)] = zeros
        sum_acc[pl.ds(v * _L, _L)] = zeros

    def _issue(c, slot):
        pltpu.async_copy(
            w_hbm.at[pl.ds(row0 + c * _CR, _CR), :], buf.at[slot], sems[slot]
        )

    def _wait(slot):
        pltpu.make_async_copy(
            w_hbm.at[pl.ds(row0, _CR), :], buf.at[slot], sems[slot]
        ).wait()

    _issue(0, 0)
    _issue(1, 1)

    def chunk_pair(c0, carry):
        for slot in range(2):
            c = c0 + slot
            _wait(slot)

            def vec_body(o, carry2):
                r = lax.shift_right_logical(o, 3)
                cb = (o & 7) * (_L * _UNROLL)
                for k in range(_UNROLL):
                    w = buf[slot, r, pl.ds(cb + k * _L, _L)]
                    bv = lax.shift_right_logical(w, 16) + lane_off
                    lv = lax.bitcast_convert_type(
                        lax.shift_left(w, 16), jnp.float32
                    )
                    plsc.addupdate_scatter(cnt_acc, [bv], ones)
                    plsc.addupdate_scatter(sum_acc, [bv], lv)
                return carry2

            lax.fori_loop(0, _VPC // _UNROLL, vec_body, 0)

            @pl.when(c + 2 < _NCHUNK)
            def _():
                _issue(c + 2, slot)
        return carry

    lax.fori_loop(0, _NCHUNK // 2, lambda i, cr: chunk_pair(i * 2, cr), 0)
    pltpu.sync_copy(cnt_acc, cnt_out.at[pl.ds(wid * _ACC, _ACC)])
    pltpu.sync_copy(sum_acc, sum_out.at[pl.ds(wid * _ACC, _ACC)])


def _combine_body(cnt_ref, sum_ref, out_ref):
    cnt = jnp.sum(cnt_ref[...], axis=0, keepdims=True)
    s = jnp.sum(sum_ref[...], axis=0, keepdims=True)
    nonempty = cnt > 0.0
    n = jnp.sum(nonempty.astype(jnp.float32))
    terms = jnp.where(nonempty, s / jnp.maximum(cnt, 1.0), 0.0)
    out_ref[0, 0] = jnp.where(n > 0.0, jnp.sum(terms) / jnp.maximum(n, 1.0), 0.0)


_combine = pl.pallas_call(
    _combine_body,
    in_specs=[
        pl.BlockSpec((_NW * _L, _PB), lambda: (0, 0)),
        pl.BlockSpec((_NW * _L, _PB), lambda: (0, 0)),
    ],
    out_specs=pl.BlockSpec(memory_space=pltpu.SMEM),
    out_shape=jax.ShapeDtypeStruct((1, 1), jnp.float32),
)


def kernel(pred, target, batch_size):
    packed = _prep(pred, target)
    cnt, s = _sc_hist(packed)
    out = _combine(cnt.reshape(_NW * _L, _PB), s.reshape(_NW * _L, _PB))
    return out[0, 0]


# speedup vs baseline: 1.6936x; 1.1604x over previous
import functools

import jax
import jax.numpy as jnp
from jax import lax
from jax.experimental import pallas as pl
from jax.experimental.pallas import tpu as pltpu
from jax.experimental.pallas import tpu_sc as plsc

_BINS = 30
_ROWS = 16384
_COLS = 1024
_TOTAL = _ROWS * _COLS

_NC = 2
_NS = 16
_L = 16
_NW = _NC * _NS
_WROWS = _ROWS // _NW
_CR = 32
_NCHUNK = _WROWS // _CR
_VPC = _CR * _COLS // _L
_UNROLL = 8

_BR = 512


def _prep_body(p_ref, t_ref, w_ref):
    p = p_ref[...]
    t = t_ref[...]
    g = jnp.abs(p - t)
    binv = jnp.minimum(jnp.floor(g * _BINS), _BINS - 1).astype(jnp.uint32)
    log_p = jnp.maximum(jnp.log(p), -100.0)
    log_1mp = jnp.maximum(jnp.log(1.0 - p), -100.0)
    l = -(t * log_p + (1.0 - t) * log_1mp)
    bits = lax.bitcast_convert_type(l, jnp.uint32) + jnp.uint32(0x8000)
    word = (binv << jnp.uint32(20)) | lax.shift_right_logical(bits, jnp.uint32(16))
    w_ref[...] = lax.bitcast_convert_type(word, jnp.int32)


_prep = pl.pallas_call(
    _prep_body,
    grid=(_ROWS // _BR,),
    in_specs=[pl.BlockSpec((_BR, _COLS), lambda i: (i, 0))] * 2,
    out_specs=pl.BlockSpec((_BR, _COLS), lambda i: (i, 0)),
    out_shape=jax.ShapeDtypeStruct((_ROWS, _COLS), jnp.int32),
)


_PB = 32
_ACC = _L * _PB


@functools.partial(
    pl.kernel,
    mesh=plsc.VectorSubcoreMesh(core_axis_name="c", subcore_axis_name="s"),
    compiler_params=pltpu.CompilerParams(needs_layout_passes=False),
    out_type=(
        jax.ShapeDtypeStruct((_NW * _ACC,), jnp.float32),
        jax.ShapeDtypeStruct((_NW * _ACC,), jnp.float32),
    ),
    scratch_types=[
        pltpu.VMEM((2, _CR, _COLS), jnp.int32),
        pltpu.VMEM((_BINS * _L,), jnp.float32),
        pltpu.VMEM((_BINS * _L,), jnp.float32),
        pltpu.VMEM((_ACC,), jnp.float32),
        pltpu.VMEM((_ACC,), jnp.float32),
        pltpu.SemaphoreType.DMA,
        pltpu.SemaphoreType.DMA,
    ],
)
def _sc_hist(w_hbm, cnt_out, sum_out, buf, cnt_acc, sum_acc, cnt_tr, sum_tr,
             sem0, sem1):
    wid = lax.axis_index("s") * _NC + lax.axis_index("c")
    row0 = wid * _WROWS
    lane = lax.iota(jnp.int32, _L)
    ones = jnp.ones((_L,), jnp.float32)
    zeros = jnp.zeros((_L,), jnp.float32)
    sems = (sem0, sem1)
    for v in range(_BINS):
        cnt_acc[pl.ds(v * _L, _L)] = zeros
        sum_acc[pl.ds(v * _L, _L)] = zeros
    for v in range(_ACC // _L):
        cnt_tr[pl.ds(v * _L, _L)] = zeros
        sum_tr[pl.ds(v * _L, _L)] = zeros

    def _issue(c, slot):
        pltpu.async_copy(
            w_hbm.at[pl.ds(row0 + c * _CR, _CR), :], buf.at[slot], sems[slot]
        )

    def _wait(slot):
        pltpu.make_async_copy(
            w_hbm.at[pl.ds(row0, _CR), :], buf.at[slot], sems[slot]
        ).wait()

    _issue(0, 0)
    _issue(1, 1)

    def chunk_pair(c0, carry):
        for slot in range(2):
            c = c0 + slot
            _wait(slot)

            def vec_body(o, carry2):
                r = lax.shift_right_logical(o, 3)
                cb = (o & 7) * (_L * _UNROLL)
                for k in range(_UNROLL):
                    w = buf[slot, r, pl.ds(cb + k * _L, _L)]
                    bv = lax.shift_right_logical(w, 16) + lane
                    lv = lax.bitcast_convert_type(
                        lax.shift_left(w, 16), jnp.float32
                    )
                    plsc.addupdate_scatter(cnt_acc, [bv], ones)
                    plsc.addupdate_scatter(sum_acc, [bv], lv)
                return carry2

            lax.fori_loop(0, _VPC // _UNROLL, vec_body, 0)

            @pl.when(c + 2 < _NCHUNK)
            def _():
                _issue(c + 2, slot)
        return carry

    lax.fori_loop(0, _NCHUNK // 2, lambda i, cr: chunk_pair(i * 2, cr), 0)
    for b in range(_BINS):
        tidx = lane * _PB + b
        plsc.store_scatter(cnt_tr, [tidx], cnt_acc[pl.ds(b * _L, _L)])
        plsc.store_scatter(sum_tr, [tidx], sum_acc[pl.ds(b * _L, _L)])
    pltpu.sync_copy(cnt_tr, cnt_out.at[pl.ds(wid * _ACC, _ACC)])
    pltpu.sync_copy(sum_tr, sum_out.at[pl.ds(wid * _ACC, _ACC)])


def _combine_body(cnt_ref, sum_ref, out_ref):
    cnt = jnp.sum(cnt_ref[...], axis=0, keepdims=True)
    s = jnp.sum(sum_ref[...], axis=0, keepdims=True)
    nonempty = cnt > 0.0
    n = jnp.sum(nonempty.astype(jnp.float32))
    terms = jnp.where(nonempty, s / jnp.maximum(cnt, 1.0), 0.0)
    out_ref[0, 0] = jnp.where(n > 0.0, jnp.sum(terms) / jnp.maximum(n, 1.0), 0.0)


_combine = pl.pallas_call(
    _combine_body,
    in_specs=[
        pl.BlockSpec((_NW * _L, _PB), lambda: (0, 0)),
        pl.BlockSpec((_NW * _L, _PB), lambda: (0, 0)),
    ],
    out_specs=pl.BlockSpec(memory_space=pltpu.SMEM),
    out_shape=jax.ShapeDtypeStruct((1, 1), jnp.float32),
)


def kernel(pred, target, batch_size):
    packed = _prep(pred, target)
    cnt, s = _sc_hist(packed)
    out = _combine(cnt.reshape(_NW * _L, _PB), s.reshape(_NW * _L, _PB))
    return out[0, 0]


# speedup vs baseline: 3.5093x; 2.0721x over previous
import functools

import jax
import jax.numpy as jnp
from jax import lax
from jax.experimental import pallas as pl
from jax.experimental.pallas import tpu as pltpu
from jax.experimental.pallas import tpu_sc as plsc

_BINS = 30
_ROWS = 16384
_COLS = 1024
_TOTAL = _ROWS * _COLS

_NC = 2
_NS = 16
_L = 16
_NW = _NC * _NS
_WROWS = _ROWS // _NW
_CR = 32
_NCHUNK = _WROWS // _CR
_VPC = _CR * _COLS // _L
_UNROLL = 8

_BR = 512


def _prep_body(p_ref, t_ref, w_ref):
    p = p_ref[...]
    t = t_ref[...]
    g = jnp.abs(p - t)
    binv = jnp.minimum(jnp.floor(g * _BINS), _BINS - 1).astype(jnp.uint32)
    log_p = jnp.maximum(jnp.log(p), -100.0)
    log_1mp = jnp.maximum(jnp.log(1.0 - p), -100.0)
    l = -(t * log_p + (1.0 - t) * log_1mp)
    bits = lax.bitcast_convert_type(l, jnp.uint32) + jnp.uint32(0x8000)
    word = (binv << jnp.uint32(20)) | lax.shift_right_logical(bits, jnp.uint32(16))
    w_ref[...] = lax.bitcast_convert_type(word, jnp.int32)


_prep = pl.pallas_call(
    _prep_body,
    grid=(_ROWS // _BR,),
    in_specs=[pl.BlockSpec((_BR, _COLS), lambda i: (i, 0))] * 2,
    out_specs=pl.BlockSpec((_BR, _COLS), lambda i: (i, 0)),
    out_shape=jax.ShapeDtypeStruct((_ROWS, _COLS), jnp.int32),
)


_PB = 32
_ACC = _L * _PB


@functools.partial(
    pl.kernel,
    mesh=plsc.VectorSubcoreMesh(core_axis_name="c", subcore_axis_name="s"),
    compiler_params=pltpu.CompilerParams(needs_layout_passes=False),
    out_type=(
        jax.ShapeDtypeStruct((_NW * _ACC,), jnp.float32),
        jax.ShapeDtypeStruct((_NW * _ACC,), jnp.float32),
    ),
    scratch_types=[
        pltpu.VMEM((2, _CR, _COLS), jnp.int32),
        pltpu.VMEM((_BINS * _L,), jnp.float32),
        pltpu.VMEM((_BINS * _L,), jnp.float32),
        pltpu.VMEM((_ACC,), jnp.float32),
        pltpu.VMEM((_ACC,), jnp.float32),
        pltpu.SemaphoreType.DMA,
        pltpu.SemaphoreType.DMA,
    ],
)
def _sc_hist(w_hbm, cnt_out, sum_out, buf, cnt_acc, sum_acc, cnt_tr, sum_tr,
             sem0, sem1):
    wid = lax.axis_index("s") * _NC + lax.axis_index("c")
    row0 = wid * _WROWS
    lane = lax.iota(jnp.int32, _L)
    ones = jnp.ones((_L,), jnp.float32)
    zeros = jnp.zeros((_L,), jnp.float32)
    sems = (sem0, sem1)
    for v in range(_BINS):
        cnt_acc[pl.ds(v * _L, _L)] = zeros
        sum_acc[pl.ds(v * _L, _L)] = zeros
    for v in range(_ACC // _L):
        cnt_tr[pl.ds(v * _L, _L)] = zeros
        sum_tr[pl.ds(v * _L, _L)] = zeros

    def _issue(c, slot):
        pltpu.async_copy(
            w_hbm.at[pl.ds(row0 + c * _CR, _CR), :], buf.at[slot], sems[slot]
        )

    def _wait(slot):
        pltpu.make_async_copy(
            w_hbm.at[pl.ds(row0, _CR), :], buf.at[slot], sems[slot]
        ).wait()

    _issue(0, 0)
    _issue(1, 1)

    def chunk_pair(c0, carry):
        for slot in range(2):
            c = c0 + slot
            _wait(slot)

            def vec_body(o, carry2):
                r = lax.shift_right_logical(o, 3)
                cb = (o & 7) * (_L * _UNROLL)
                ws = [
                    buf[slot, r, pl.ds(cb + k * _L, _L)] for k in range(_UNROLL)
                ]
                bvs = [lax.shift_right_logical(w, 16) + lane for w in ws]
                lvs = [
                    lax.bitcast_convert_type(lax.shift_left(w, 16), jnp.float32)
                    for w in ws
                ]
                for k in range(_UNROLL):
                    plsc.addupdate_scatter(cnt_acc, [bvs[k]], ones)
                for k in range(_UNROLL):
                    plsc.addupdate_scatter(sum_acc, [bvs[k]], lvs[k])
                return carry2

            lax.fori_loop(0, _VPC // _UNROLL, vec_body, 0)

            @pl.when(c + 2 < _NCHUNK)
            def _():
                _issue(c + 2, slot)
        return carry

    lax.fori_loop(0, _NCHUNK // 2, lambda i, cr: chunk_pair(i * 2, cr), 0)
    for b in range(_BINS):
        tidx = lane * _PB + b
        plsc.store_scatter(cnt_tr, [tidx], cnt_acc[pl.ds(b * _L, _L)])
        plsc.store_scatter(sum_tr, [tidx], sum_acc[pl.ds(b * _L, _L)])
    pltpu.sync_copy(cnt_tr, cnt_out.at[pl.ds(wid * _ACC, _ACC)])
    pltpu.sync_copy(sum_tr, sum_out.at[pl.ds(wid * _ACC, _ACC)])


def _combine_body(cnt_ref, sum_ref, out_ref):
    cnt = jnp.sum(cnt_ref[...], axis=0, keepdims=True)
    s = jnp.sum(sum_ref[...], axis=0, keepdims=True)
    nonempty = cnt > 0.0
    n = jnp.sum(nonempty.astype(jnp.float32))
    terms = jnp.where(nonempty, s / jnp.maximum(cnt, 1.0), 0.0)
    out_ref[0, 0] = jnp.where(n > 0.0, jnp.sum(terms) / jnp.maximum(n, 1.0), 0.0)


_combine = pl.pallas_call(
    _combine_body,
    in_specs=[
        pl.BlockSpec((_NW * _L, _PB), lambda: (0, 0)),
        pl.BlockSpec((_NW * _L, _PB), lambda: (0, 0)),
    ],
    out_specs=pl.BlockSpec(memory_space=pltpu.SMEM),
    out_shape=jax.ShapeDtypeStruct((1, 1), jnp.float32),
)


def kernel(pred, target, batch_size):
    packed = _prep(pred, target)
    cnt, s = _sc_hist(packed)
    out = _combine(cnt.reshape(_NW * _L, _PB), s.reshape(_NW * _L, _PB))
    return out[0, 0]


# speedup vs baseline: 3.6708x; 1.0460x over previous
import functools

import jax
import jax.numpy as jnp
from jax import lax
from jax.experimental import pallas as pl
from jax.experimental.pallas import tpu as pltpu
from jax.experimental.pallas import tpu_sc as plsc

_BINS = 30
_ROWS = 16384
_COLS = 1024
_TOTAL = _ROWS * _COLS

_NC = 2
_NS = 16
_L = 16
_NW = _NC * _NS
_WROWS = _ROWS // _NW
_CR = 32
_NCHUNK = _WROWS // _CR
_VPC = _CR * _COLS // _L
_UNROLL = 16

_BR = 512


def _prep_body(p_ref, t_ref, w_ref):
    p = p_ref[...]
    t = t_ref[...]
    g = jnp.abs(p - t)
    binv = jnp.minimum(jnp.floor(g * _BINS), _BINS - 1).astype(jnp.uint32)
    log_p = jnp.maximum(jnp.log(p), -100.0)
    log_1mp = jnp.maximum(jnp.log(1.0 - p), -100.0)
    l = -(t * log_p + (1.0 - t) * log_1mp)
    bits = lax.bitcast_convert_type(l, jnp.uint32) + jnp.uint32(0x8000)
    word = (binv << jnp.uint32(20)) | lax.shift_right_logical(bits, jnp.uint32(16))
    w_ref[...] = lax.bitcast_convert_type(word, jnp.int32)


_prep = pl.pallas_call(
    _prep_body,
    grid=(_ROWS // _BR,),
    in_specs=[pl.BlockSpec((_BR, _COLS), lambda i: (i, 0))] * 2,
    out_specs=pl.BlockSpec((_BR, _COLS), lambda i: (i, 0)),
    out_shape=jax.ShapeDtypeStruct((_ROWS, _COLS), jnp.int32),
)


_PB = 32
_ACC = _L * _PB


@functools.partial(
    pl.kernel,
    mesh=plsc.VectorSubcoreMesh(core_axis_name="c", subcore_axis_name="s"),
    compiler_params=pltpu.CompilerParams(needs_layout_passes=False),
    out_type=(
        jax.ShapeDtypeStruct((_NW * _ACC,), jnp.float32),
        jax.ShapeDtypeStruct((_NW * _ACC,), jnp.float32),
    ),
    scratch_types=[
        pltpu.VMEM((2, _CR, _COLS), jnp.int32),
        pltpu.VMEM((_BINS * _L,), jnp.float32),
        pltpu.VMEM((_BINS * _L,), jnp.float32),
        pltpu.VMEM((_ACC,), jnp.float32),
        pltpu.VMEM((_ACC,), jnp.float32),
        pltpu.SemaphoreType.DMA,
        pltpu.SemaphoreType.DMA,
    ],
)
def _sc_hist(w_hbm, cnt_out, sum_out, buf, cnt_acc, sum_acc, cnt_tr, sum_tr,
             sem0, sem1):
    wid = lax.axis_index("s") * _NC + lax.axis_index("c")
    row0 = wid * _WROWS
    lane = lax.iota(jnp.int32, _L)
    ones = jnp.ones((_L,), jnp.float32)
    zeros = jnp.zeros((_L,), jnp.float32)
    sems = (sem0, sem1)
    for v in range(_BINS):
        cnt_acc[pl.ds(v * _L, _L)] = zeros
        sum_acc[pl.ds(v * _L, _L)] = zeros
    for v in range(_ACC // _L):
        cnt_tr[pl.ds(v * _L, _L)] = zeros
        sum_tr[pl.ds(v * _L, _L)] = zeros

    def _issue(c, slot):
        pltpu.async_copy(
            w_hbm.at[pl.ds(row0 + c * _CR, _CR), :], buf.at[slot], sems[slot]
        )

    def _wait(slot):
        pltpu.make_async_copy(
            w_hbm.at[pl.ds(row0, _CR), :], buf.at[slot], sems[slot]
        ).wait()

    _issue(0, 0)
    _issue(1, 1)

    def chunk_pair(c0, carry):
        for slot in range(2):
            c = c0 + slot
            _wait(slot)

            def vec_body(o, carry2):
                gpr = _COLS // (_L * _UNROLL)
                r = lax.shift_right_logical(o, gpr.bit_length() - 1)
                cb = (o & (gpr - 1)) * (_L * _UNROLL)
                ws = [
                    buf[slot, r, pl.ds(cb + k * _L, _L)] for k in range(_UNROLL)
                ]
                bvs = [lax.shift_right_logical(w, 16) + lane for w in ws]
                lvs = [
                    lax.bitcast_convert_type(lax.shift_left(w, 16), jnp.float32)
                    for w in ws
                ]
                for k in range(_UNROLL):
                    plsc.addupdate_scatter(cnt_acc, [bvs[k]], ones)
                for k in range(_UNROLL):
                    plsc.addupdate_scatter(sum_acc, [bvs[k]], lvs[k])
                return carry2

            lax.fori_loop(0, _VPC // _UNROLL, vec_body, 0)

            @pl.when(c + 2 < _NCHUNK)
            def _():
                _issue(c + 2, slot)
        return carry

    lax.fori_loop(0, _NCHUNK // 2, lambda i, cr: chunk_pair(i * 2, cr), 0)
    for b in range(_BINS):
        tidx = lane * _PB + b
        plsc.store_scatter(cnt_tr, [tidx], cnt_acc[pl.ds(b * _L, _L)])
        plsc.store_scatter(sum_tr, [tidx], sum_acc[pl.ds(b * _L, _L)])
    pltpu.sync_copy(cnt_tr, cnt_out.at[pl.ds(wid * _ACC, _ACC)])
    pltpu.sync_copy(sum_tr, sum_out.at[pl.ds(wid * _ACC, _ACC)])


def _combine_body(cnt_ref, sum_ref, out_ref):
    cnt = jnp.sum(cnt_ref[...], axis=0, keepdims=True)
    s = jnp.sum(sum_ref[...], axis=0, keepdims=True)
    nonempty = cnt > 0.0
    n = jnp.sum(nonempty.astype(jnp.float32))
    terms = jnp.where(nonempty, s / jnp.maximum(cnt, 1.0), 0.0)
    out_ref[0, 0] = jnp.where(n > 0.0, jnp.sum(terms) / jnp.maximum(n, 1.0), 0.0)


_combine = pl.pallas_call(
    _combine_body,
    in_specs=[
        pl.BlockSpec((_NW * _L, _PB), lambda: (0, 0)),
        pl.BlockSpec((_NW * _L, _PB), lambda: (0, 0)),
    ],
    out_specs=pl.BlockSpec(memory_space=pltpu.SMEM),
    out_shape=jax.ShapeDtypeStruct((1, 1), jnp.float32),
)


def kernel(pred, target, batch_size):
    packed = _prep(pred, target)
    cnt, s = _sc_hist(packed)
    out = _combine(cnt.reshape(_NW * _L, _PB), s.reshape(_NW * _L, _PB))
    return out[0, 0]


# speedup vs baseline: 4.1523x; 1.1312x over previous
import functools

import jax
import jax.numpy as jnp
from jax import lax
from jax.experimental import pallas as pl
from jax.experimental.pallas import tpu as pltpu
from jax.experimental.pallas import tpu_sc as plsc

_BINS = 30
_ROWS = 16384
_COLS = 1024

_P = 4
_GROWS = _ROWS // _P

_NC = 2
_NS = 16
_L = 16
_NW = _NC * _NS
_WROWS = _GROWS // _NW
_CR = 32
_NCHUNK = _WROWS // _CR
_VPC = _CR * _COLS // _L
_UNROLL = 16

_BR = 512
_GB = _GROWS // _BR

_PB = 32
_ACC = _L * _PB


def _prep_body(p_ref, t_ref, w_ref):
    p = p_ref[...]
    t = t_ref[...]
    g = jnp.abs(p - t)
    binv = jnp.minimum(jnp.floor(g * _BINS), _BINS - 1).astype(jnp.uint32)
    log_p = jnp.maximum(jnp.log(p), -100.0)
    log_1mp = jnp.maximum(jnp.log(1.0 - p), -100.0)
    l = -(t * log_p + (1.0 - t) * log_1mp)
    bits = lax.bitcast_convert_type(l, jnp.uint32) + jnp.uint32(0x8000)
    word = (binv << jnp.uint32(20)) | lax.shift_right_logical(bits, jnp.uint32(16))
    w_ref[...] = lax.bitcast_convert_type(word, jnp.int32)


def _make_prep(grp):
    return pl.pallas_call(
        _prep_body,
        grid=(_GB,),
        in_specs=[
            pl.BlockSpec((_BR, _COLS), lambda i, g=grp: (i + g * _GB, 0))
        ] * 2,
        out_specs=pl.BlockSpec((_BR, _COLS), lambda i: (i, 0)),
        out_shape=jax.ShapeDtypeStruct((_GROWS, _COLS), jnp.int32),
    )


_preps = [_make_prep(g) for g in range(_P)]


@functools.partial(
    pl.kernel,
    mesh=plsc.VectorSubcoreMesh(core_axis_name="c", subcore_axis_name="s"),
    compiler_params=pltpu.CompilerParams(needs_layout_passes=False),
    out_type=(
        jax.ShapeDtypeStruct((_NW * _ACC,), jnp.float32),
        jax.ShapeDtypeStruct((_NW * _ACC,), jnp.float32),
    ),
    scratch_types=[
        pltpu.VMEM((2, _CR, _COLS), jnp.int32),
        pltpu.VMEM((_BINS * _L,), jnp.float32),
        pltpu.VMEM((_BINS * _L,), jnp.float32),
        pltpu.VMEM((_ACC,), jnp.float32),
        pltpu.VMEM((_ACC,), jnp.float32),
        pltpu.SemaphoreType.DMA,
        pltpu.SemaphoreType.DMA,
    ],
)
def _sc_hist(w_hbm, cnt_out, sum_out, buf, cnt_acc, sum_acc, cnt_tr, sum_tr,
             sem0, sem1):
    wid = lax.axis_index("s") * _NC + lax.axis_index("c")
    row0 = wid * _WROWS
    lane = lax.iota(jnp.int32, _L)
    ones = jnp.ones((_L,), jnp.float32)
    zeros = jnp.zeros((_L,), jnp.float32)
    sems = (sem0, sem1)
    for v in range(_BINS):
        cnt_acc[pl.ds(v * _L, _L)] = zeros
        sum_acc[pl.ds(v * _L, _L)] = zeros
    for v in range(_ACC // _L):
        cnt_tr[pl.ds(v * _L, _L)] = zeros
        sum_tr[pl.ds(v * _L, _L)] = zeros

    def _issue(c, slot):
        pltpu.async_copy(
            w_hbm.at[pl.ds(row0 + c * _CR, _CR), :], buf.at[slot], sems[slot]
        )

    def _wait(slot):
        pltpu.make_async_copy(
            w_hbm.at[pl.ds(row0, _CR), :], buf.at[slot], sems[slot]
        ).wait()

    _issue(0, 0)
    _issue(1, 1)

    def chunk_pair(c0, carry):
        for slot in range(2):
            c = c0 + slot
            _wait(slot)

            def vec_body(o, carry2):
                gpr = _COLS // (_L * _UNROLL)
                r = lax.shift_right_logical(o, gpr.bit_length() - 1)
                cb = (o & (gpr - 1)) * (_L * _UNROLL)
                ws = [
                    buf[slot, r, pl.ds(cb + k * _L, _L)] for k in range(_UNROLL)
                ]
                bvs = [lax.shift_right_logical(w, 16) + lane for w in ws]
                lvs = [
                    lax.bitcast_convert_type(lax.shift_left(w, 16), jnp.float32)
                    for w in ws
                ]
                for k in range(_UNROLL):
                    plsc.addupdate_scatter(cnt_acc, [bvs[k]], ones)
                for k in range(_UNROLL):
                    plsc.addupdate_scatter(sum_acc, [bvs[k]], lvs[k])
                return carry2

            lax.fori_loop(0, _VPC // _UNROLL, vec_body, 0)

            @pl.when(c + 2 < _NCHUNK)
            def _():
                _issue(c + 2, slot)
        return carry

    lax.fori_loop(0, _NCHUNK // 2, lambda i, cr: chunk_pair(i * 2, cr), 0)
    for b in range(_BINS):
        tidx = lane * _PB + b
        plsc.store_scatter(cnt_tr, [tidx], cnt_acc[pl.ds(b * _L, _L)])
        plsc.store_scatter(sum_tr, [tidx], sum_acc[pl.ds(b * _L, _L)])
    pltpu.sync_copy(cnt_tr, cnt_out.at[pl.ds(wid * _ACC, _ACC)])
    pltpu.sync_copy(sum_tr, sum_out.at[pl.ds(wid * _ACC, _ACC)])


def _combine_body(*refs):
    cnt_refs = refs[:_P]
    sum_refs = refs[_P : 2 * _P]
    out_ref = refs[2 * _P]
    cnt = sum(jnp.sum(r[...], axis=0, keepdims=True) for r in cnt_refs)
    s = sum(jnp.sum(r[...], axis=0, keepdims=True) for r in sum_refs)
    nonempty = cnt > 0.0
    n = jnp.sum(nonempty.astype(jnp.float32))
    terms = jnp.where(nonempty, s / jnp.maximum(cnt, 1.0), 0.0)
    out_ref[0, 0] = jnp.where(n > 0.0, jnp.sum(terms) / jnp.maximum(n, 1.0), 0.0)


_combine = pl.pallas_call(
    _combine_body,
    in_specs=[pl.BlockSpec((_NW * _L, _PB), lambda: (0, 0))] * (2 * _P),
    out_specs=pl.BlockSpec(memory_space=pltpu.SMEM),
    out_shape=jax.ShapeDtypeStruct((1, 1), jnp.float32),
)


def kernel(pred, target, batch_size):
    cnts, sums = [], []
    for g in range(_P):
        packed = _preps[g](pred, target)
        cnt, s = _sc_hist(packed)
        cnts.append(cnt.reshape(_NW * _L, _PB))
        sums.append(s.reshape(_NW * _L, _PB))
    out = _combine(*cnts, *sums)
    return out[0, 0]


# speedup vs baseline: 4.1555x; 1.0008x over previous
import functools

import jax
import jax.numpy as jnp
from jax import lax
from jax.experimental import pallas as pl
from jax.experimental.pallas import tpu as pltpu
from jax.experimental.pallas import tpu_sc as plsc

_BINS = 30
_ROWS = 16384
_COLS = 1024

_P = 2
_GROWS = _ROWS // _P

_NC = 2
_NS = 16
_L = 16
_NW = _NC * _NS
_WROWS = _GROWS // _NW
_CR = 32
_NCHUNK = _WROWS // _CR
_VPC = _CR * _COLS // _L
_UNROLL = 16

_BR = 512
_GB = _GROWS // _BR

_PB = 32
_ACC = _L * _PB


def _prep_body(p_ref, t_ref, w_ref):
    p = p_ref[...]
    t = t_ref[...]
    g = jnp.abs(p - t)
    binv = jnp.minimum(jnp.floor(g * _BINS), _BINS - 1).astype(jnp.uint32)
    log_p = jnp.maximum(jnp.log(p), -100.0)
    log_1mp = jnp.maximum(jnp.log(1.0 - p), -100.0)
    l = -(t * log_p + (1.0 - t) * log_1mp)
    bits = lax.bitcast_convert_type(l, jnp.uint32) + jnp.uint32(0x8000)
    word = (binv << jnp.uint32(20)) | lax.shift_right_logical(bits, jnp.uint32(16))
    w_ref[...] = lax.bitcast_convert_type(word, jnp.int32)


def _make_prep(grp):
    return pl.pallas_call(
        _prep_body,
        grid=(_GB,),
        in_specs=[
            pl.BlockSpec((_BR, _COLS), lambda i, g=grp: (i + g * _GB, 0))
        ] * 2,
        out_specs=pl.BlockSpec((_BR, _COLS), lambda i: (i, 0)),
        out_shape=jax.ShapeDtypeStruct((_GROWS, _COLS), jnp.int32),
    )


_preps = [_make_prep(g) for g in range(_P)]


@functools.partial(
    pl.kernel,
    mesh=plsc.VectorSubcoreMesh(core_axis_name="c", subcore_axis_name="s"),
    compiler_params=pltpu.CompilerParams(needs_layout_passes=False),
    out_type=(
        jax.ShapeDtypeStruct((_NW * _ACC,), jnp.float32),
        jax.ShapeDtypeStruct((_NW * _ACC,), jnp.float32),
    ),
    scratch_types=[
        pltpu.VMEM((2, _CR, _COLS), jnp.int32),
        pltpu.VMEM((_BINS * _L,), jnp.float32),
        pltpu.VMEM((_BINS * _L,), jnp.float32),
        pltpu.VMEM((_ACC,), jnp.float32),
        pltpu.VMEM((_ACC,), jnp.float32),
        pltpu.SemaphoreType.DMA,
        pltpu.SemaphoreType.DMA,
    ],
)
def _sc_hist(w_hbm, cnt_out, sum_out, buf, cnt_acc, sum_acc, cnt_tr, sum_tr,
             sem0, sem1):
    wid = lax.axis_index("s") * _NC + lax.axis_index("c")
    row0 = wid * _WROWS
    lane = lax.iota(jnp.int32, _L)
    ones = jnp.ones((_L,), jnp.float32)
    zeros = jnp.zeros((_L,), jnp.float32)
    sems = (sem0, sem1)
    for v in range(_BINS):
        cnt_acc[pl.ds(v * _L, _L)] = zeros
        sum_acc[pl.ds(v * _L, _L)] = zeros
    for v in range(_ACC // _L):
        cnt_tr[pl.ds(v * _L, _L)] = zeros
        sum_tr[pl.ds(v * _L, _L)] = zeros

    def _issue(c, slot):
        pltpu.async_copy(
            w_hbm.at[pl.ds(row0 + c * _CR, _CR), :], buf.at[slot], sems[slot]
        )

    def _wait(slot):
        pltpu.make_async_copy(
            w_hbm.at[pl.ds(row0, _CR), :], buf.at[slot], sems[slot]
        ).wait()

    _issue(0, 0)
    _issue(1, 1)

    def chunk_pair(c0, carry):
        for slot in range(2):
            c = c0 + slot
            _wait(slot)

            def vec_body(o, carry2):
                gpr = _COLS // (_L * _UNROLL)
                r = lax.shift_right_logical(o, gpr.bit_length() - 1)
                cb = (o & (gpr - 1)) * (_L * _UNROLL)
                ws = [
                    buf[slot, r, pl.ds(cb + k * _L, _L)] for k in range(_UNROLL)
                ]
                bvs = [lax.shift_right_logical(w, 16) + lane for w in ws]
                lvs = [
                    lax.bitcast_convert_type(lax.shift_left(w, 16), jnp.float32)
                    for w in ws
                ]
                for k in range(_UNROLL):
                    plsc.addupdate_scatter(cnt_acc, [bvs[k]], ones)
                for k in range(_UNROLL):
                    plsc.addupdate_scatter(sum_acc, [bvs[k]], lvs[k])
                return carry2

            lax.fori_loop(0, _VPC // _UNROLL, vec_body, 0)

            @pl.when(c + 2 < _NCHUNK)
            def _():
                _issue(c + 2, slot)
        return carry

    lax.fori_loop(0, _NCHUNK // 2, lambda i, cr: chunk_pair(i * 2, cr), 0)
    for b in range(_BINS):
        tidx = lane * _PB + b
        plsc.store_scatter(cnt_tr, [tidx], cnt_acc[pl.ds(b * _L, _L)])
        plsc.store_scatter(sum_tr, [tidx], sum_acc[pl.ds(b * _L, _L)])
    pltpu.sync_copy(cnt_tr, cnt_out.at[pl.ds(wid * _ACC, _ACC)])
    pltpu.sync_copy(sum_tr, sum_out.at[pl.ds(wid * _ACC, _ACC)])


def _combine_body(*refs):
    cnt_refs = refs[:_P]
    sum_refs = refs[_P : 2 * _P]
    out_ref = refs[2 * _P]
    cnt = sum(jnp.sum(r[...], axis=0, keepdims=True) for r in cnt_refs)
    s = sum(jnp.sum(r[...], axis=0, keepdims=True) for r in sum_refs)
    nonempty = cnt > 0.0
    n = jnp.sum(nonempty.astype(jnp.float32))
    terms = jnp.where(nonempty, s / jnp.maximum(cnt, 1.0), 0.0)
    out_ref[0, 0] = jnp.where(n > 0.0, jnp.sum(terms) / jnp.maximum(n, 1.0), 0.0)


_combine = pl.pallas_call(
    _combine_body,
    in_specs=[pl.BlockSpec((_NW * _L, _PB), lambda: (0, 0))] * (2 * _P),
    out_specs=pl.BlockSpec(memory_space=pltpu.SMEM),
    out_shape=jax.ShapeDtypeStruct((1, 1), jnp.float32),
)


def kernel(pred, target, batch_size):
    cnts, sums = [], []
    for g in range(_P):
        packed = _preps[g](pred, target)
        cnt, s = _sc_hist(packed)
        cnts.append(cnt.reshape(_NW * _L, _PB))
        sums.append(s.reshape(_NW * _L, _PB))
    out = _combine(*cnts, *sums)
    return out[0, 0]
